# baseline (speedup 1.0000x reference)
import functools

import jax
import jax.numpy as jnp
from jax import lax
from jax.experimental import pallas as pl
from jax.experimental.pallas import tpu as pltpu
from jax.experimental.pallas import tpu_sc as plsc

N = 1024
F = 128
OUT = 128
K = 9
L = 16
NW = 32
NSC = 256
DPW = NSC // NW
NTC = N - NSC
BI = 8
CHUNK = 16
NEGB = float(-3e38)
NEGF = float(-3e38)

_GD = lax.GatherDimensionNumbers(offset_dims=(), collapsed_slice_dims=(0,),
                                 start_index_map=(0,))


def _gather1d(v, idx):
    return lax.gather(v, idx[:, None], _GD, slice_sizes=(1,),
                      mode=lax.GatherScatterMode.PROMISE_IN_BOUNDS)


def _xlane_max(v):
    for sh in (1, 2, 4, 8):
        idx = (jnp.arange(L, dtype=jnp.int32) + sh) % L
        v = jnp.maximum(v, _gather1d(v, idx))
    return v


def _sc_topk_kernel():
    mesh = plsc.VectorSubcoreMesh(core_axis_name="c", subcore_axis_name="s")

    @functools.partial(
        pl.kernel,
        mesh=mesh,
        out_type=jax.ShapeDtypeStruct((NSC, F * L), jnp.float32),
        scratch_types=[
            pltpu.VMEM((DPW, N), jnp.float32),
            pltpu.VMEM((2, N), jnp.float32),
            pltpu.VMEM((DPW, F * L), jnp.float32),
            pltpu.SemaphoreType.DMA((2,)),
        ],
    )
    def sc_topk(supT_hbm, feaT_hbm, out_hbm, a_v, x2_v, o_v, sems):
        wid = lax.axis_index("s") * 2 + lax.axis_index("c")
        base = wid * DPW
        pltpu.sync_copy(supT_hbm.at[pl.ds(base, DPW)], a_v)
        lanes = jnp.arange(L, dtype=jnp.int32)
        pltpu.make_async_copy(feaT_hbm.at[0], x2_v.at[0], sems.at[0]).start()

        def j_loop(j, _):
            cur = j % 2
            x_v = x2_v.at[cur]
            pltpu.make_async_copy(feaT_hbm.at[j], x_v, sems.at[cur]).wait()

            @pl.when(j + 1 < F)
            def _():
                pltpu.make_async_copy(feaT_hbm.at[j + 1],
                                      x2_v.at[(j + 1) % 2],
                                      sems.at[(j + 1) % 2]).start()

            def i_loop(i, _):
                def c_loop(c, R):
                    out = list(R)
                    for u in range(2):
                        a = a_v[i, pl.ds((2 * c + u) * L, L)]
                        x = x_v[pl.ds((2 * c + u) * L, L)]
                        v = a * x
                        for t in range(K):
                            hi = jnp.maximum(out[t], v)
                            v = jnp.minimum(out[t], v)
                            out[t] = hi
                    return tuple(out)

                R0 = tuple(jnp.full((L,), NEGF, dtype=jnp.float32)
                           for _ in range(K))
                R = lax.fori_loop(0, N // L // 2, c_loop, R0)

                def t_loop(t, carry):
                    cand, out_vec = carry
                    m = cand[0]
                    for q in range(1, K):
                        m = jnp.maximum(m, cand[q])
                    m = _xlane_max(m)
                    out_vec = jnp.where(lanes == t, m, out_vec)
                    cand = tuple(jnp.where(c < m, c, NEGF) for c in cand)
                    return (cand, out_vec)

                _, out_vec = lax.fori_loop(
                    0, K, t_loop, (R, jnp.full((L,), 0.0, dtype=jnp.float32)))
                o_v[i, pl.ds(j * L, L)] = out_vec
                return 0

            lax.fori_loop(0, DPW, i_loop, 0)
            return 0

        lax.fori_loop(0, F, j_loop, 0)
        pltpu.sync_copy(o_v, out_hbm.at[pl.ds(base, DPW)])

    return sc_topk


def _mm_body(tk_ref, w_ref, b_ref, out_ref):
    acc = jnp.dot(tk_ref[:], w_ref[:], preferred_element_type=jnp.float32)
    out_ref[:] = jnp.maximum(acc + b_ref[:], 0.0)


def _tc_body(adj_ref, fea_ref, dsel_ref, w_ref, b_ref, out_ref, topk_ref,
             pan_ref):
    a8t = adj_ref[0]
    pan_ref[:] = jax.lax.dot_general(
        a8t, dsel_ref[:], (((0,), (0,)), ((), ())),
        preferred_element_type=jnp.float32).astype(jnp.bfloat16)
    for d in range(BI):

        def s1_body(c, R, d=d):
            v = (pan_ref[pl.ds(c * CHUNK, CHUNK), d * F : (d + 1) * F]
                 * fea_ref[pl.ds(c * CHUNK, CHUNK), :])
            out = []
            for t in range(K):
                hi = jnp.maximum(R[t], v)
                v = jnp.minimum(R[t], v)
                out.append(hi)
            return tuple(out)

        R0 = tuple(jnp.full((CHUNK, F), NEGB, dtype=jnp.bfloat16)
                   for _ in range(K))
        R = jax.lax.fori_loop(0, N // CHUNK, s1_body, R0, unroll=16)
        c32 = jnp.concatenate([r.astype(jnp.float32) for r in R], axis=0)
        cand = jnp.concatenate(
            [jnp.maximum(c32[t * CHUNK : t * CHUNK + 8, :],
                         c32[(8 - t) * CHUNK + 8 : (8 - t) * CHUNK + 16, :])
             for t in range(K)], axis=0)
        ids = jax.lax.broadcasted_iota(jnp.int32, (K * 8, F), 0)
        keys = jax.lax.bitcast_convert_type(
            jax.lax.bitcast_convert_type(cand, jnp.int32) | ids, jnp.float32)

        def s2_body(t, m, d=d):
            val = jax.lax.bitcast_convert_type(
                jax.lax.bitcast_convert_type(m, jnp.int32) & (~0xFF),
                jnp.float32)
            topk_ref[pl.ds(t * BI + d, 1), :] = val
            return jnp.max(jnp.where(keys < m, keys, NEGB), axis=0,
                           keepdims=True)

        m0 = jnp.max(keys, axis=0, keepdims=True)
        jax.lax.fori_loop(0, K, s2_body, m0)

    acc = jnp.zeros((BI, OUT), dtype=jnp.float32)
    for t in range(K):
        acc += jnp.dot(topk_ref[t * BI : (t + 1) * BI, :], w_ref[t],
                       preferred_element_type=jnp.float32)
    out_ref[:] = jnp.maximum(acc + b_ref[:], 0.0)


@jax.jit
def kernel(inputs, support, W, b):
    supT = support.T
    feaT = inputs.T
    b2 = b.reshape(1, OUT)

    topk_sc = _sc_topk_kernel()(supT[:NSC], feaT)

    fea = inputs.astype(jnp.bfloat16)
    support3 = supT[NSC:].astype(jnp.bfloat16).reshape(NTC // BI, BI, N)
    dsel = (jnp.arange(BI, dtype=jnp.int32)[:, None]
            == (jnp.arange(BI * F, dtype=jnp.int32) // F)[None, :]
            ).astype(jnp.bfloat16)
    out_tc = pl.pallas_call(
        _tc_body,
        grid=(NTC // BI,),
        in_specs=[
            pl.BlockSpec((1, BI, N), lambda ib: (ib, 0, 0)),
            pl.BlockSpec((N, F), lambda ib: (0, 0)),
            pl.BlockSpec((BI, BI * F), lambda ib: (0, 0)),
            pl.BlockSpec((K, F, OUT), lambda ib: (0, 0, 0)),
            pl.BlockSpec((1, OUT), lambda ib: (0, 0)),
        ],
        out_specs=pl.BlockSpec((BI, OUT), lambda ib: (ib, 0)),
        out_shape=jax.ShapeDtypeStruct((NTC, OUT), jnp.float32),
        scratch_shapes=[
            pltpu.VMEM((K * BI, F), jnp.float32),
            pltpu.VMEM((N, BI * F), jnp.bfloat16),
        ],
    )(support3, fea, dsel, W, b2)

    wpad = jnp.pad(jnp.transpose(W, (1, 0, 2)),
                   ((0, 0), (0, L - K), (0, 0))).reshape(F * L, OUT)
    out_sc = pl.pallas_call(
        _mm_body,
        grid=(NSC // 64,),
        in_specs=[
            pl.BlockSpec((64, F * L), lambda ib: (ib, 0)),
            pl.BlockSpec((F * L, OUT), lambda ib: (0, 0)),
            pl.BlockSpec((1, OUT), lambda ib: (0, 0)),
        ],
        out_specs=pl.BlockSpec((64, OUT), lambda ib: (ib, 0)),
        out_shape=jax.ShapeDtypeStruct((NSC, OUT), jnp.float32),
    )(topk_sc, wpad, b2)

    return jnp.concatenate([out_sc, out_tc], axis=0)


# speedup vs baseline: 1.0241x; 1.0241x over previous
import functools

import jax
import jax.numpy as jnp
from jax import lax
from jax.experimental import pallas as pl
from jax.experimental.pallas import tpu as pltpu
from jax.experimental.pallas import tpu_sc as plsc

N = 1024
F = 128
OUT = 128
K = 9
L = 16
NW = 32
NSC = 256
DPW = NSC // NW
NTC = N - NSC
BI = 8
CHUNK = 16
NEGB = float(-3e38)
NEGF = float(-3e38)

_GD = lax.GatherDimensionNumbers(offset_dims=(), collapsed_slice_dims=(0,),
                                 start_index_map=(0,))


def _gather1d(v, idx):
    return lax.gather(v, idx[:, None], _GD, slice_sizes=(1,),
                      mode=lax.GatherScatterMode.PROMISE_IN_BOUNDS)


def _xlane_max(v):
    for sh in (1, 2, 4, 8):
        idx = (jnp.arange(L, dtype=jnp.int32) + sh) % L
        v = jnp.maximum(v, _gather1d(v, idx))
    return v


def _sc_topk_kernel():
    mesh = plsc.VectorSubcoreMesh(core_axis_name="c", subcore_axis_name="s")

    @functools.partial(
        pl.kernel,
        mesh=mesh,
        out_type=jax.ShapeDtypeStruct((NSC, F * L), jnp.float32),
        scratch_types=[
            pltpu.VMEM((DPW, N), jnp.float32),
            pltpu.VMEM((2, N), jnp.float32),
            pltpu.VMEM((DPW, F * L), jnp.float32),
            pltpu.SemaphoreType.DMA((2,)),
        ],
    )
    def sc_topk(supT_hbm, feaT_hbm, out_hbm, a_v, x2_v, o_v, sems):
        wid = lax.axis_index("s") * 2 + lax.axis_index("c")
        base = wid * DPW
        pltpu.sync_copy(supT_hbm.at[pl.ds(base, DPW)], a_v)
        lanes = jnp.arange(L, dtype=jnp.int32)
        pltpu.make_async_copy(feaT_hbm.at[0], x2_v.at[0], sems.at[0]).start()

        def j_loop(j, _):
            cur = j % 2
            x_v = x2_v.at[cur]
            pltpu.make_async_copy(feaT_hbm.at[j], x_v, sems.at[cur]).wait()

            @pl.when(j + 1 < F)
            def _():
                pltpu.make_async_copy(feaT_hbm.at[j + 1],
                                      x2_v.at[(j + 1) % 2],
                                      sems.at[(j + 1) % 2]).start()

            def i_loop(i, _):
                def c_loop(c, R):
                    out = list(R)
                    for u in range(2):
                        a = a_v[i, pl.ds((2 * c + u) * L, L)]
                        x = x_v[pl.ds((2 * c + u) * L, L)]
                        v = a * x
                        for t in range(K):
                            hi = jnp.maximum(out[t], v)
                            v = jnp.minimum(out[t], v)
                            out[t] = hi
                    return tuple(out)

                R0 = tuple(jnp.full((L,), NEGF, dtype=jnp.float32)
                           for _ in range(K))
                R = lax.fori_loop(0, N // L // 2, c_loop, R0)

                def t_loop(t, carry):
                    cand, out_vec = carry
                    m = cand[0]
                    for q in range(1, K):
                        m = jnp.maximum(m, cand[q])
                    m = _xlane_max(m)
                    out_vec = jnp.where(lanes == t, m, out_vec)
                    cand = tuple(jnp.where(c < m, c, NEGF) for c in cand)
                    return (cand, out_vec)

                _, out_vec = lax.fori_loop(
                    0, K, t_loop, (R, jnp.full((L,), 0.0, dtype=jnp.float32)))
                o_v[i, pl.ds(j * L, L)] = out_vec
                return 0

            lax.fori_loop(0, DPW, i_loop, 0)
            return 0

        lax.fori_loop(0, F, j_loop, 0)
        pltpu.sync_copy(o_v, out_hbm.at[pl.ds(base, DPW)])

    return sc_topk


def _mm_body(tk_ref, w_ref, b_ref, out_ref):
    acc = jnp.dot(tk_ref[:], w_ref[:], preferred_element_type=jnp.float32)
    out_ref[:] = jnp.maximum(acc + b_ref[:], 0.0)


def _tc_body(adj_ref, fea_ref, dsel_ref, w_ref, b_ref, out_ref, topk_ref,
             pan_ref):
    a8t = adj_ref[0]
    pan_ref[:] = jax.lax.dot_general(
        a8t, dsel_ref[:], (((0,), (0,)), ((), ())),
        preferred_element_type=jnp.float32).astype(jnp.bfloat16)
    for d in range(BI):

        def s1_body(c, R, d=d):
            v = (pan_ref[pl.ds(c * CHUNK, CHUNK), d * F : (d + 1) * F]
                 * fea_ref[pl.ds(c * CHUNK, CHUNK), :])
            out = []
            for t in range(K):
                hi = jnp.maximum(R[t], v)
                v = jnp.minimum(R[t], v)
                out.append(hi)
            return tuple(out)

        R0 = tuple(jnp.full((CHUNK, F), NEGB, dtype=jnp.bfloat16)
                   for _ in range(K))
        R = jax.lax.fori_loop(0, N // CHUNK, s1_body, R0, unroll=32)
        c32 = jnp.concatenate([r.astype(jnp.float32) for r in R], axis=0)
        cand = jnp.concatenate(
            [jnp.maximum(c32[t * CHUNK : t * CHUNK + 8, :],
                         c32[(8 - t) * CHUNK + 8 : (8 - t) * CHUNK + 16, :])
             for t in range(K)], axis=0)
        ids = jax.lax.broadcasted_iota(jnp.int32, (K * 8, F), 0)
        keys = jax.lax.bitcast_convert_type(
            jax.lax.bitcast_convert_type(cand, jnp.int32) | ids, jnp.float32)

        def s2_body(t, m, d=d):
            val = jax.lax.bitcast_convert_type(
                jax.lax.bitcast_convert_type(m, jnp.int32) & (~0xFF),
                jnp.float32)
            topk_ref[pl.ds(t * BI + d, 1), :] = val
            return jnp.max(jnp.where(keys < m, keys, NEGB), axis=0,
                           keepdims=True)

        m0 = jnp.max(keys, axis=0, keepdims=True)
        jax.lax.fori_loop(0, K, s2_body, m0)

    acc = jnp.zeros((BI, OUT), dtype=jnp.float32)
    for t in range(K):
        acc += jnp.dot(topk_ref[t * BI : (t + 1) * BI, :], w_ref[t],
                       preferred_element_type=jnp.float32)
    out_ref[:] = jnp.maximum(acc + b_ref[:], 0.0)


@jax.jit
def kernel(inputs, support, W, b):
    supT = support.T
    feaT = inputs.T
    b2 = b.reshape(1, OUT)

    topk_sc = _sc_topk_kernel()(supT[:NSC], feaT)

    fea = inputs.astype(jnp.bfloat16)
    support3 = supT[NSC:].astype(jnp.bfloat16).reshape(NTC // BI, BI, N)
    dsel = (jnp.arange(BI, dtype=jnp.int32)[:, None]
            == (jnp.arange(BI * F, dtype=jnp.int32) // F)[None, :]
            ).astype(jnp.bfloat16)
    out_tc = pl.pallas_call(
        _tc_body,
        grid=(NTC // BI,),
        in_specs=[
            pl.BlockSpec((1, BI, N), lambda ib: (ib, 0, 0)),
            pl.BlockSpec((N, F), lambda ib: (0, 0)),
            pl.BlockSpec((BI, BI * F), lambda ib: (0, 0)),
            pl.BlockSpec((K, F, OUT), lambda ib: (0, 0, 0)),
            pl.BlockSpec((1, OUT), lambda ib: (0, 0)),
        ],
        out_specs=pl.BlockSpec((BI, OUT), lambda ib: (ib, 0)),
        out_shape=jax.ShapeDtypeStruct((NTC, OUT), jnp.float32),
        scratch_shapes=[
            pltpu.VMEM((K * BI, F), jnp.float32),
            pltpu.VMEM((N, BI * F), jnp.bfloat16),
        ],
    )(support3, fea, dsel, W, b2)

    wpad = jnp.pad(jnp.transpose(W, (1, 0, 2)),
                   ((0, 0), (0, L - K), (0, 0))).reshape(F * L, OUT)
    out_sc = pl.pallas_call(
        _mm_body,
        grid=(NSC // 64,),
        in_specs=[
            pl.BlockSpec((64, F * L), lambda ib: (ib, 0)),
            pl.BlockSpec((F * L, OUT), lambda ib: (0, 0)),
            pl.BlockSpec((1, OUT), lambda ib: (0, 0)),
        ],
        out_specs=pl.BlockSpec((64, OUT), lambda ib: (ib, 0)),
        out_shape=jax.ShapeDtypeStruct((NSC, OUT), jnp.float32),
    )(topk_sc, wpad, b2)

    return jnp.concatenate([out_sc, out_tc], axis=0)
